# baseline (speedup 1.0000x reference)
import jax
import jax.numpy as jnp
from jax.experimental import pallas as pl
from jax.experimental.pallas import tpu as pltpu

_HIDDEN = 2048
_EXPERTS = 64
_BLK = 1024
_NSPLIT = 2


def _router_block(*refs):
    xs = refs[:_NSPLIT]
    w_ref = refs[_NSPLIT]
    o_ref = refs[_NSPLIT + 1]
    w = w_ref[...].astype(jnp.bfloat16)
    dn = (((1,), (1,)), ((), ()))
    for s in range(_NSPLIT):
        x = xs[s][0].astype(jnp.bfloat16)
        o_ref[s] = jax.lax.dot_general(
            x, w, dn, preferred_element_type=jnp.float32)


def kernel(hidden_states, weight):
    hs = hidden_states.reshape(-1, _HIDDEN)
    n = hs.shape[0]
    rows = n // _NSPLIT
    hs3 = hs.reshape(_NSPLIT, rows, _HIDDEN)
    steps = rows // _BLK

    def x_spec(s):
        return pl.BlockSpec((1, _BLK, _HIDDEN), lambda i, s=s: (s, i, 0))

    out = pl.pallas_call(
        _router_block,
        grid=(steps,),
        in_specs=[x_spec(s) for s in range(_NSPLIT)] + [
            pl.BlockSpec((_EXPERTS, _HIDDEN), lambda i: (0, 0)),
        ],
        out_specs=pl.BlockSpec((_NSPLIT, _BLK, _EXPERTS), lambda i: (0, i, 0)),
        out_shape=jax.ShapeDtypeStruct((_NSPLIT, rows, _EXPERTS), jnp.float32),
        compiler_params=pltpu.CompilerParams(
            dimension_semantics=(pltpu.PARALLEL,),
            vmem_limit_bytes=100 * 1024 * 1024,
        ),
    )(*([hs3] * _NSPLIT), weight)
    return out.reshape(n, _EXPERTS)


# speedup vs baseline: 1.0311x; 1.0311x over previous
import jax
import jax.numpy as jnp
from jax.experimental import pallas as pl
from jax.experimental.pallas import tpu as pltpu

_HIDDEN = 2048
_EXPERTS = 64
_BLK = 512
_NSPLIT = 2


def _router_block(*refs):
    xs = refs[:_NSPLIT]
    w_ref = refs[_NSPLIT]
    o_ref = refs[_NSPLIT + 1]
    w = w_ref[...].astype(jnp.bfloat16)
    dn = (((1,), (1,)), ((), ()))
    for s in range(_NSPLIT):
        x = xs[s][0].astype(jnp.bfloat16)
        o_ref[s] = jax.lax.dot_general(
            x, w, dn, preferred_element_type=jnp.float32)


def kernel(hidden_states, weight):
    hs = hidden_states.reshape(-1, _HIDDEN)
    n = hs.shape[0]
    rows = n // _NSPLIT
    hs3 = hs.reshape(_NSPLIT, rows, _HIDDEN)
    steps = rows // _BLK

    def x_spec(s):
        return pl.BlockSpec((1, _BLK, _HIDDEN), lambda i, s=s: (s, i, 0))

    out = pl.pallas_call(
        _router_block,
        grid=(steps,),
        in_specs=[x_spec(s) for s in range(_NSPLIT)] + [
            pl.BlockSpec((_EXPERTS, _HIDDEN), lambda i: (0, 0)),
        ],
        out_specs=pl.BlockSpec((_NSPLIT, _BLK, _EXPERTS), lambda i: (0, i, 0)),
        out_shape=jax.ShapeDtypeStruct((_NSPLIT, rows, _EXPERTS), jnp.float32),
        compiler_params=pltpu.CompilerParams(
            dimension_semantics=(pltpu.PARALLEL,),
            vmem_limit_bytes=100 * 1024 * 1024,
        ),
    )(*([hs3] * _NSPLIT), weight)
    return out.reshape(n, _EXPERTS)
